# baseline (speedup 1.0000x reference)
import functools

import jax
import jax.numpy as jnp
from jax import lax
from jax.experimental import pallas as pl
from jax.experimental.pallas import tpu as pltpu
from jax.experimental.pallas import tpu_sc as plsc

D_MODEL = 1024
NUM_EXPERTS = 8
TOP_K = 2
N_TOKENS = 4096
N_PAIRS = N_TOKENS * TOP_K

TILE = 256
HALF_TOKENS = N_TOKENS // 2
HALF_PAIRS = HALF_TOKENS * TOP_K
P_PAD_H = ((HALF_PAIRS + NUM_EXPERTS * (TILE - 1)) + TILE - 1) // TILE * TILE
NT_H = P_PAD_H // TILE

SC_CORES = 2
SC_SUBCORES = 16
NUM_WORKERS = SC_CORES * SC_SUBCORES
TOK_PW_H = HALF_TOKENS // NUM_WORKERS


ROUTE_BLK = 256
ROUTE_GRID = N_TOKENS // ROUTE_BLK


def _route_kernel(x_ref, gw_ref, gb_ref,
                  e1_ref, e2_ref, r0_ref, r1_ref, g1_ref, g2_ref,
                  cnt_ref, carry_v):
    i = pl.program_id(0)
    n = ROUTE_BLK
    xblk = x_ref[...]
    logits = jnp.dot(xblk, gw_ref[...],
                     preferred_element_type=jnp.float32) + gb_ref[...]
    col = lax.broadcasted_iota(jnp.int32, (n, NUM_EXPERTS), 1)
    m1 = jnp.max(logits, axis=1, keepdims=True)
    a1 = jnp.min(jnp.where(logits == m1, col, NUM_EXPERTS), axis=1, keepdims=True)
    neg = jnp.float32(-jnp.inf)
    l2 = jnp.where(col == a1, neg, logits)
    m2 = jnp.max(l2, axis=1, keepdims=True)
    a2 = jnp.min(jnp.where(l2 == m2, col, NUM_EXPERTS), axis=1, keepdims=True)
    oh1 = (col == a1).astype(jnp.float32)
    oh2 = (col == a2).astype(jnp.float32)
    colsum = jnp.sum(oh1 + oh2, axis=0, keepdims=True)

    @pl.when((i == 0) | (i == ROUTE_GRID // 2))
    def _():
        carry_v[...] = jnp.zeros_like(colsum)

    rr = lax.broadcasted_iota(jnp.int32, (n, n), 0)
    cc = lax.broadcasted_iota(jnp.int32, (n, n), 1)
    tri = (rr >= cc).astype(jnp.bfloat16)
    c1 = jnp.dot(tri, oh1.astype(jnp.bfloat16),
                 preferred_element_type=jnp.float32)
    c2 = jnp.dot(tri, oh2.astype(jnp.bfloat16),
                 preferred_element_type=jnp.float32)
    carry = carry_v[...]
    r0 = carry + (c1 - oh1) + (c2 - oh2)
    r1 = carry + c1 + c2 - oh2
    e1_ref[...] = a1
    e2_ref[...] = a2
    r0_ref[...] = jnp.sum(oh1 * r0, axis=1, keepdims=True).astype(jnp.int32)
    r1_ref[...] = jnp.sum(oh2 * r1, axis=1, keepdims=True).astype(jnp.int32)
    g1 = 1.0 / (1.0 + jnp.exp(m2 - m1))
    g1_ref[...] = g1
    g2_ref[...] = 1.0 - g1
    carry_v[...] = carry + colsum
    cnt_ref[...] = (carry + colsum).astype(jnp.int32).reshape(1, 1, NUM_EXPERTS)


def _routing(x, gate_W, gate_b):
    gb2 = gate_b.reshape(1, NUM_EXPERTS)
    ospec = pl.BlockSpec((ROUTE_BLK, 1), lambda i: (i, 0))
    outs = [
        jax.ShapeDtypeStruct((N_TOKENS, 1), jnp.int32),
        jax.ShapeDtypeStruct((N_TOKENS, 1), jnp.int32),
        jax.ShapeDtypeStruct((N_TOKENS, 1), jnp.int32),
        jax.ShapeDtypeStruct((N_TOKENS, 1), jnp.int32),
        jax.ShapeDtypeStruct((N_TOKENS, 1), jnp.float32),
        jax.ShapeDtypeStruct((N_TOKENS, 1), jnp.float32),
        jax.ShapeDtypeStruct((2, 1, NUM_EXPERTS), jnp.int32),
    ]
    return pl.pallas_call(
        _route_kernel,
        grid=(ROUTE_GRID,),
        in_specs=[
            pl.BlockSpec((ROUTE_BLK, D_MODEL), lambda i: (i, 0)),
            pl.BlockSpec((D_MODEL, NUM_EXPERTS), lambda i: (0, 0)),
            pl.BlockSpec((1, NUM_EXPERTS), lambda i: (0, 0)),
        ],
        out_specs=[ospec, ospec, ospec, ospec, ospec, ospec,
                   pl.BlockSpec((1, 1, NUM_EXPERTS),
                                lambda i: (i // (ROUTE_GRID // 2), 0, 0))],
        out_shape=outs,
        scratch_shapes=[pltpu.VMEM((1, NUM_EXPERTS), jnp.float32)],
        compiler_params=pltpu.CompilerParams(
            dimension_semantics=("arbitrary",)),
    )(x, gate_W, gb2)


SCHUNK = 32
N_SCHUNK = TOK_PW_H // SCHUNK


def _make_scatter(tok_base):
    mesh = plsc.VectorSubcoreMesh(core_axis_name="c", subcore_axis_name="s")

    @functools.partial(
        pl.kernel,
        mesh=mesh,
        out_type=jax.ShapeDtypeStruct((P_PAD_H, D_MODEL), jnp.float32),
        scratch_types=[
            pltpu.VMEM((16,), jnp.int32),
            pltpu.VMEM((TOK_PW_H,), jnp.int32),
            pltpu.VMEM((TOK_PW_H,), jnp.int32),
            pltpu.VMEM((N_SCHUNK, SCHUNK), jnp.int32),
            pltpu.VMEM((N_SCHUNK, SCHUNK), jnp.int32),
            pltpu.VMEM((SCHUNK, D_MODEL), jnp.float32),
            pltpu.VMEM((SCHUNK, D_MODEL), jnp.float32),
            pltpu.SemaphoreType.DMA,
            pltpu.SemaphoreType.DMA,
            pltpu.SemaphoreType.DMA,
            pltpu.SemaphoreType.DMA,
        ],
    )
    def scatter(xp_hbm, e1_hbm, e2_hbm, r0_hbm, r1_hbm, off_hbm, xs_hbm,
                off_v, e_v, r_v, i0_m, i1_m, rows_a, rows_b,
                sem0, sem1, sem2, sem3):
        wid = lax.axis_index("s") * SC_CORES + lax.axis_index("c")
        base = tok_base + wid * TOK_PW_H
        wsl = pl.ds(base, TOK_PW_H)
        pltpu.sync_copy(off_hbm, off_v)
        ov = off_v[...]

        def dest_vec(ev, rv):
            b = jnp.zeros((16,), jnp.int32)
            for e in range(NUM_EXPERTS):
                b = jnp.where(ev == e, ov[e], b)
            return rv + b

        ca = pltpu.async_copy(xp_hbm.at[pl.ds(base, SCHUNK)], rows_a, sem0)

        pltpu.sync_copy(e1_hbm.at[wsl], e_v)
        pltpu.sync_copy(r0_hbm.at[wsl], r_v)
        for j in range(TOK_PW_H // 16):
            s16 = pl.ds(j * 16, 16)
            i0_m[j // (SCHUNK // 16),
                 pl.ds((j % (SCHUNK // 16)) * 16, 16)] = dest_vec(
                     e_v[s16], r_v[s16])
        pltpu.sync_copy(e2_hbm.at[wsl], e_v)
        pltpu.sync_copy(r1_hbm.at[wsl], r_v)
        for j in range(TOK_PW_H // 16):
            s16 = pl.ds(j * 16, 16)
            i1_m[j // (SCHUNK // 16),
                 pl.ds((j % (SCHUNK // 16)) * 16, 16)] = dest_vec(
                     e_v[s16], r_v[s16])

        bufs = (rows_a, rows_b)
        sems = (sem0, sem1)
        ca.wait()
        for c in range(N_SCHUNK):
            if c + 1 < N_SCHUNK:
                cn = pltpu.async_copy(
                    xp_hbm.at[pl.ds(base + (c + 1) * SCHUNK, SCHUNK)],
                    bufs[(c + 1) % 2], sems[(c + 1) % 2])
            c0 = pltpu.async_copy(bufs[c % 2], xs_hbm.at[i0_m.at[c]], sem2)
            c1 = pltpu.async_copy(bufs[c % 2], xs_hbm.at[i1_m.at[c]], sem3)
            c0.wait()
            c1.wait()
            if c + 1 < N_SCHUNK:
                cn.wait()

    return scatter


def _gmm_kernel(te_ref, xs_ref, w_ref, b_ref, y_ref):
    y_ref[...] = jnp.dot(xs_ref[...].astype(jnp.bfloat16), w_ref[0],
                         preferred_element_type=jnp.float32) + b_ref[0]


def _gmm(tile_expert, xs, expert_W_bf16, expert_b):
    grid_spec = pltpu.PrefetchScalarGridSpec(
        num_scalar_prefetch=1,
        grid=(NT_H,),
        in_specs=[
            pl.BlockSpec((TILE, D_MODEL), lambda i, te: (i, 0)),
            pl.BlockSpec((1, D_MODEL, D_MODEL), lambda i, te: (te[i], 0, 0)),
            pl.BlockSpec((1, 1, D_MODEL), lambda i, te: (te[i], 0, 0)),
        ],
        out_specs=pl.BlockSpec((TILE, D_MODEL), lambda i, te: (i, 0)),
    )
    return pl.pallas_call(
        _gmm_kernel,
        grid_spec=grid_spec,
        out_shape=jax.ShapeDtypeStruct((P_PAD_H, D_MODEL), jnp.float32),
        compiler_params=pltpu.CompilerParams(
            dimension_semantics=("arbitrary",)),
    )(tile_expert, xs, expert_W_bf16,
      expert_b.reshape(NUM_EXPERTS, 1, D_MODEL))


GX = 32
N_GCHUNK = TOK_PW_H // GX


def _make_gather2(tok_base):
    mesh = plsc.VectorSubcoreMesh(core_axis_name="c", subcore_axis_name="s")

    @functools.partial(
        pl.kernel,
        mesh=mesh,
        out_type=[
            jax.ShapeDtypeStruct((HALF_TOKENS, D_MODEL), jnp.float32),
            jax.ShapeDtypeStruct((HALF_TOKENS, D_MODEL), jnp.float32),
        ],
        scratch_types=[
            pltpu.VMEM((16,), jnp.int32),
            pltpu.VMEM((TOK_PW_H,), jnp.int32),
            pltpu.VMEM((TOK_PW_H,), jnp.int32),
            pltpu.VMEM((N_GCHUNK, GX), jnp.int32),
            pltpu.VMEM((N_GCHUNK, GX), jnp.int32),
            pltpu.VMEM((GX, D_MODEL), jnp.float32),
            pltpu.VMEM((GX, D_MODEL), jnp.float32),
            pltpu.SemaphoreType.DMA,
            pltpu.SemaphoreType.DMA,
        ],
    )
    def gather2(y_hbm, e1_hbm, e2_hbm, r0_hbm, r1_hbm, off_hbm,
                y0_hbm, y1_hbm,
                off_v, e_v, r_v, i0_m, i1_m, r0b_v, r1b_v, sem0, sem1):
        wid = lax.axis_index("s") * SC_CORES + lax.axis_index("c")
        base = tok_base + wid * TOK_PW_H
        obase = wid * TOK_PW_H
        wsl = pl.ds(base, TOK_PW_H)
        pltpu.sync_copy(off_hbm, off_v)
        ov = off_v[...]

        def dest_vec(ev, rv):
            b = jnp.zeros((16,), jnp.int32)
            for e in range(NUM_EXPERTS):
                b = jnp.where(ev == e, ov[e], b)
            return rv + b

        pltpu.sync_copy(e1_hbm.at[wsl], e_v)
        pltpu.sync_copy(r0_hbm.at[wsl], r_v)
        for j in range(TOK_PW_H // 16):
            s16 = pl.ds(j * 16, 16)
            i0_m[j // (GX // 16), pl.ds((j % (GX // 16)) * 16, 16)] = dest_vec(
                e_v[s16], r_v[s16])
        pltpu.sync_copy(e2_hbm.at[wsl], e_v)
        pltpu.sync_copy(r1_hbm.at[wsl], r_v)
        for j in range(TOK_PW_H // 16):
            s16 = pl.ds(j * 16, 16)
            i1_m[j // (GX // 16), pl.ds((j % (GX // 16)) * 16, 16)] = dest_vec(
                e_v[s16], r_v[s16])

        for c in range(N_GCHUNK):
            tb = obase + c * GX
            c0 = pltpu.async_copy(y_hbm.at[i0_m.at[c]], r0b_v, sem0)
            c1 = pltpu.async_copy(y_hbm.at[i1_m.at[c]], r1b_v, sem1)
            c0.wait()
            c1.wait()
            pltpu.sync_copy(r0b_v, y0_hbm.at[pl.ds(tb, GX)])
            pltpu.sync_copy(r1b_v, y1_hbm.at[pl.ds(tb, GX)])

    return gather2


_scatter_fn = functools.lru_cache(maxsize=None)(_make_scatter)
_gather2_fn = functools.lru_cache(maxsize=None)(_make_gather2)


def _combine_kernel(y0_ref, y1_ref, g1_ref, g2_ref, o_ref):
    o_ref[...] = y0_ref[...] * g1_ref[...] + y1_ref[...] * g2_ref[...]


def _tc_combine(y0, y1, g1, g2):
    grid = 4
    blk = HALF_TOKENS // grid
    return pl.pallas_call(
        _combine_kernel,
        grid=(grid,),
        in_specs=[
            pl.BlockSpec((blk, D_MODEL), lambda i: (i, 0)),
            pl.BlockSpec((blk, D_MODEL), lambda i: (i, 0)),
            pl.BlockSpec((blk, 1), lambda i: (i, 0)),
            pl.BlockSpec((blk, 1), lambda i: (i, 0)),
        ],
        out_specs=pl.BlockSpec((blk, D_MODEL), lambda i: (i, 0)),
        out_shape=jax.ShapeDtypeStruct((HALF_TOKENS, D_MODEL), jnp.float32),
    )(y0, y1, g1, g2)


def kernel(x, gate_W, gate_b, expert_W, expert_b):
    e1, e2, r0, r1, g1, g2, counts = _routing(x, gate_W, gate_b)

    tile_starts = jnp.arange(NT_H, dtype=jnp.int32) * TILE

    def _half_meta(cnt):
        padded = (cnt + TILE - 1) // TILE * TILE
        off = jnp.concatenate([jnp.zeros((1,), jnp.int32),
                               jnp.cumsum(padded)[:-1].astype(jnp.int32)])
        off16 = jnp.concatenate([off, jnp.zeros((16 - NUM_EXPERTS,),
                                                jnp.int32)])
        te = jnp.sum((tile_starts[:, None] >= off[None, :]).astype(jnp.int32),
                     axis=1) - 1
        return off16, jnp.clip(te, 0, NUM_EXPERTS - 1)

    off16_a, te_a = _half_meta(counts[0, 0])
    off16_b, te_b = _half_meta(counts[1, 0])

    expert_W_bf16 = expert_W.astype(jnp.bfloat16)
    e1f = e1.reshape(-1)
    e2f = e2.reshape(-1)
    r0f = r0.reshape(-1)
    r1f = r1.reshape(-1)

    xs_a = _scatter_fn(0)(x, e1f, e2f, r0f, r1f, off16_a)
    xs_b = _scatter_fn(HALF_TOKENS)(x, e1f, e2f, r0f, r1f, off16_b)
    y_a = _gmm(te_a, xs_a, expert_W_bf16, expert_b)
    y_b = _gmm(te_b, xs_b, expert_W_bf16, expert_b)
    y0_a, y1_a = _gather2_fn(0)(y_a, e1f, e2f, r0f, r1f, off16_a)
    y0_b, y1_b = _gather2_fn(HALF_TOKENS)(y_b, e1f, e2f, r0f, r1f, off16_b)
    out_a = _tc_combine(y0_a, y1_a, g1[:HALF_TOKENS], g2[:HALF_TOKENS])
    out_b = _tc_combine(y0_b, y1_b, g1[HALF_TOKENS:], g2[HALF_TOKENS:])
    return jnp.concatenate([out_a, out_b], axis=0)


# speedup vs baseline: 1.1805x; 1.1805x over previous
import functools

import jax
import jax.numpy as jnp
from jax import lax
from jax.experimental import pallas as pl
from jax.experimental.pallas import tpu as pltpu
from jax.experimental.pallas import tpu_sc as plsc

D_MODEL = 1024
NUM_EXPERTS = 8
TOP_K = 2
N_TOKENS = 4096
N_PAIRS = N_TOKENS * TOP_K

TILE = 256
P_PAD = ((N_PAIRS + NUM_EXPERTS * (TILE - 1)) + TILE - 1) // TILE * TILE
NUM_TILES = P_PAD // TILE

SC_CORES = 2
SC_SUBCORES = 16
NUM_WORKERS = SC_CORES * SC_SUBCORES
TOK_PER_WORKER = N_TOKENS // NUM_WORKERS
CHUNK = 16
GCHUNK = 32
PD = D_MODEL // 2


ROUTE_BLK = 256
ROUTE_GRID = N_TOKENS // ROUTE_BLK


def _route_kernel(x_ref, gw_ref, gb_ref,
                  e1_ref, e2_ref, r0_ref, r1_ref, g1_ref, g2_ref,
                  cnt_ref, carry_v):
    i = pl.program_id(0)
    n = ROUTE_BLK
    xblk = x_ref[...]
    logits = jnp.dot(xblk, gw_ref[...],
                     preferred_element_type=jnp.float32) + gb_ref[...]
    col = lax.broadcasted_iota(jnp.int32, (n, NUM_EXPERTS), 1)
    m1 = jnp.max(logits, axis=1, keepdims=True)
    a1 = jnp.min(jnp.where(logits == m1, col, NUM_EXPERTS), axis=1, keepdims=True)
    neg = jnp.float32(-jnp.inf)
    l2 = jnp.where(col == a1, neg, logits)
    m2 = jnp.max(l2, axis=1, keepdims=True)
    a2 = jnp.min(jnp.where(l2 == m2, col, NUM_EXPERTS), axis=1, keepdims=True)
    oh1 = (col == a1).astype(jnp.float32)
    oh2 = (col == a2).astype(jnp.float32)
    colsum = jnp.sum(oh1 + oh2, axis=0, keepdims=True)

    @pl.when(i == 0)
    def _():
        carry_v[...] = jnp.zeros_like(colsum)

    rr = lax.broadcasted_iota(jnp.int32, (n, n), 0)
    cc = lax.broadcasted_iota(jnp.int32, (n, n), 1)
    tri = (rr >= cc).astype(jnp.bfloat16)
    c1 = jnp.dot(tri, oh1.astype(jnp.bfloat16),
                 preferred_element_type=jnp.float32)
    c2 = jnp.dot(tri, oh2.astype(jnp.bfloat16),
                 preferred_element_type=jnp.float32)
    carry = carry_v[...]
    r0 = carry + (c1 - oh1) + (c2 - oh2)
    r1 = carry + c1 + c2 - oh2
    e1_ref[...] = a1.reshape(n)
    e2_ref[...] = a2.reshape(n)
    r0_ref[...] = jnp.sum(oh1 * r0, axis=1).astype(jnp.int32)
    r1_ref[...] = jnp.sum(oh2 * r1, axis=1).astype(jnp.int32)
    g1 = 1.0 / (1.0 + jnp.exp(m2 - m1))
    g1_ref[...] = g1
    g2_ref[...] = 1.0 - g1
    carry_v[...] = carry + colsum
    cnt_ref[...] = (carry + colsum).astype(jnp.int32)


def _routing(x, gate_W, gate_b):
    gb2 = gate_b.reshape(1, NUM_EXPERTS)
    ospec = pl.BlockSpec((ROUTE_BLK, 1), lambda i: (i, 0))
    ospec1 = pl.BlockSpec((ROUTE_BLK,), lambda i: (i,))
    outs = [
        jax.ShapeDtypeStruct((N_TOKENS,), jnp.int32),
        jax.ShapeDtypeStruct((N_TOKENS,), jnp.int32),
        jax.ShapeDtypeStruct((N_TOKENS,), jnp.int32),
        jax.ShapeDtypeStruct((N_TOKENS,), jnp.int32),
        jax.ShapeDtypeStruct((N_TOKENS, 1), jnp.float32),
        jax.ShapeDtypeStruct((N_TOKENS, 1), jnp.float32),
        jax.ShapeDtypeStruct((1, NUM_EXPERTS), jnp.int32),
    ]
    return pl.pallas_call(
        _route_kernel,
        grid=(ROUTE_GRID,),
        in_specs=[
            pl.BlockSpec((ROUTE_BLK, D_MODEL), lambda i: (i, 0)),
            pl.BlockSpec((D_MODEL, NUM_EXPERTS), lambda i: (0, 0)),
            pl.BlockSpec((1, NUM_EXPERTS), lambda i: (0, 0)),
        ],
        out_specs=[ospec1, ospec1, ospec1, ospec1, ospec, ospec,
                   pl.BlockSpec((1, NUM_EXPERTS), lambda i: (0, 0))],
        out_shape=outs,
        scratch_shapes=[pltpu.VMEM((1, NUM_EXPERTS), jnp.float32)],
        compiler_params=pltpu.CompilerParams(
            dimension_semantics=("arbitrary",)),
    )(x, gate_W, gb2)


SCHUNK = 32
N_SCHUNK = TOK_PER_WORKER // SCHUNK


def _make_scatter():
    mesh = plsc.VectorSubcoreMesh(core_axis_name="c", subcore_axis_name="s")

    @functools.partial(
        pl.kernel,
        mesh=mesh,
        out_type=jax.ShapeDtypeStruct((P_PAD, D_MODEL), jnp.float32),
        scratch_types=[
            pltpu.VMEM((16,), jnp.int32),
            pltpu.VMEM((TOK_PER_WORKER,), jnp.int32),
            pltpu.VMEM((TOK_PER_WORKER,), jnp.int32),
            pltpu.VMEM((N_SCHUNK, SCHUNK), jnp.int32),
            pltpu.VMEM((N_SCHUNK, SCHUNK), jnp.int32),
            pltpu.VMEM((SCHUNK, D_MODEL), jnp.float32),
            pltpu.VMEM((SCHUNK, D_MODEL), jnp.float32),
            pltpu.SemaphoreType.DMA,
            pltpu.SemaphoreType.DMA,
            pltpu.SemaphoreType.DMA,
            pltpu.SemaphoreType.DMA,
        ],
    )
    def scatter(xp_hbm, e1_hbm, e2_hbm, r0_hbm, r1_hbm, off_hbm, xs_hbm,
                off_v, e_v, r_v, i0_m, i1_m, rows_a, rows_b,
                sem0, sem1, sem2, sem3):
        wid = lax.axis_index("s") * SC_CORES + lax.axis_index("c")
        base = wid * TOK_PER_WORKER
        wsl = pl.ds(base, TOK_PER_WORKER)
        pltpu.sync_copy(off_hbm, off_v)
        ov = off_v[...]

        def dest_vec(ev, rv):
            b = jnp.zeros((16,), jnp.int32)
            for e in range(NUM_EXPERTS):
                b = jnp.where(ev == e, ov[e], b)
            return rv + b

        ca = pltpu.async_copy(xp_hbm.at[pl.ds(base, SCHUNK)], rows_a, sem0)

        pltpu.sync_copy(e1_hbm.at[wsl], e_v)
        pltpu.sync_copy(r0_hbm.at[wsl], r_v)
        for j in range(TOK_PER_WORKER // 16):
            s16 = pl.ds(j * 16, 16)
            i0_m[j // (SCHUNK // 16),
                 pl.ds((j % (SCHUNK // 16)) * 16, 16)] = dest_vec(
                     e_v[s16], r_v[s16])
        pltpu.sync_copy(e2_hbm.at[wsl], e_v)
        pltpu.sync_copy(r1_hbm.at[wsl], r_v)
        for j in range(TOK_PER_WORKER // 16):
            s16 = pl.ds(j * 16, 16)
            i1_m[j // (SCHUNK // 16),
                 pl.ds((j % (SCHUNK // 16)) * 16, 16)] = dest_vec(
                     e_v[s16], r_v[s16])

        bufs = (rows_a, rows_b)
        sems = (sem0, sem1)
        ca.wait()
        for c in range(N_SCHUNK):
            if c + 1 < N_SCHUNK:
                cn = pltpu.async_copy(
                    xp_hbm.at[pl.ds(base + (c + 1) * SCHUNK, SCHUNK)],
                    bufs[(c + 1) % 2], sems[(c + 1) % 2])
            c0 = pltpu.async_copy(bufs[c % 2], xs_hbm.at[i0_m.at[c]], sem2)
            c1 = pltpu.async_copy(bufs[c % 2], xs_hbm.at[i1_m.at[c]], sem3)
            c0.wait()
            c1.wait()
            if c + 1 < N_SCHUNK:
                cn.wait()

    return scatter


def _gmm_kernel(te_ref, xs_ref, w_ref, b_ref, y_ref):
    y_ref[...] = jnp.dot(xs_ref[...].astype(jnp.bfloat16), w_ref[0],
                         preferred_element_type=jnp.float32) + b_ref[0]


def _gmm(tile_expert, xs, expert_W_bf16, expert_b):
    grid_spec = pltpu.PrefetchScalarGridSpec(
        num_scalar_prefetch=1,
        grid=(NUM_TILES,),
        in_specs=[
            pl.BlockSpec((TILE, D_MODEL), lambda i, te: (i, 0)),
            pl.BlockSpec((1, D_MODEL, D_MODEL), lambda i, te: (te[i], 0, 0)),
            pl.BlockSpec((1, 1, D_MODEL), lambda i, te: (te[i], 0, 0)),
        ],
        out_specs=pl.BlockSpec((TILE, D_MODEL), lambda i, te: (i, 0)),
    )
    return pl.pallas_call(
        _gmm_kernel,
        grid_spec=grid_spec,
        out_shape=jax.ShapeDtypeStruct((P_PAD, D_MODEL), jnp.float32),
        compiler_params=pltpu.CompilerParams(
            dimension_semantics=("arbitrary",)),
    )(tile_expert, xs, expert_W_bf16,
      expert_b.reshape(NUM_EXPERTS, 1, D_MODEL))


GX = 32
N_GCHUNK = TOK_PER_WORKER // GX


def _make_gather2():
    mesh = plsc.VectorSubcoreMesh(core_axis_name="c", subcore_axis_name="s")

    @functools.partial(
        pl.kernel,
        mesh=mesh,
        out_type=[
            jax.ShapeDtypeStruct((N_TOKENS, D_MODEL), jnp.float32),
            jax.ShapeDtypeStruct((N_TOKENS, D_MODEL), jnp.float32),
        ],
        scratch_types=[
            pltpu.VMEM((16,), jnp.int32),
            pltpu.VMEM((TOK_PER_WORKER,), jnp.int32),
            pltpu.VMEM((TOK_PER_WORKER,), jnp.int32),
            pltpu.VMEM((N_GCHUNK, GX), jnp.int32),
            pltpu.VMEM((N_GCHUNK, GX), jnp.int32),
            pltpu.VMEM((GX, D_MODEL), jnp.float32),
            pltpu.VMEM((GX, D_MODEL), jnp.float32),
            pltpu.SemaphoreType.DMA,
            pltpu.SemaphoreType.DMA,
        ],
    )
    def gather2(y_hbm, e1_hbm, e2_hbm, r0_hbm, r1_hbm, off_hbm,
                y0_hbm, y1_hbm,
                off_v, e_v, r_v, i0_m, i1_m, r0b_v, r1b_v, sem0, sem1):
        wid = lax.axis_index("s") * SC_CORES + lax.axis_index("c")
        base = wid * TOK_PER_WORKER
        wsl = pl.ds(base, TOK_PER_WORKER)
        pltpu.sync_copy(off_hbm, off_v)
        ov = off_v[...]

        def dest_vec(ev, rv):
            b = jnp.zeros((16,), jnp.int32)
            for e in range(NUM_EXPERTS):
                b = jnp.where(ev == e, ov[e], b)
            return rv + b

        pltpu.sync_copy(e1_hbm.at[wsl], e_v)
        pltpu.sync_copy(r0_hbm.at[wsl], r_v)
        for j in range(TOK_PER_WORKER // 16):
            s16 = pl.ds(j * 16, 16)
            i0_m[j // (GX // 16), pl.ds((j % (GX // 16)) * 16, 16)] = dest_vec(
                e_v[s16], r_v[s16])
        pltpu.sync_copy(e2_hbm.at[wsl], e_v)
        pltpu.sync_copy(r1_hbm.at[wsl], r_v)
        for j in range(TOK_PER_WORKER // 16):
            s16 = pl.ds(j * 16, 16)
            i1_m[j // (GX // 16), pl.ds((j % (GX // 16)) * 16, 16)] = dest_vec(
                e_v[s16], r_v[s16])

        for c in range(N_GCHUNK):
            tb = base + c * GX
            c0 = pltpu.async_copy(y_hbm.at[i0_m.at[c]], r0b_v, sem0)
            c1 = pltpu.async_copy(y_hbm.at[i1_m.at[c]], r1b_v, sem1)
            c0.wait()
            c1.wait()
            pltpu.sync_copy(r0b_v, y0_hbm.at[pl.ds(tb, GX)])
            pltpu.sync_copy(r1b_v, y1_hbm.at[pl.ds(tb, GX)])

    return gather2


_scatter_fn = functools.lru_cache(maxsize=None)(_make_scatter)
_gather2_fn = functools.lru_cache(maxsize=None)(_make_gather2)


def _combine_kernel(y0_ref, y1_ref, g1_ref, g2_ref, o_ref):
    o_ref[...] = y0_ref[...] * g1_ref[...] + y1_ref[...] * g2_ref[...]


def _tc_combine(y0, y1, g1, g2):
    grid = 8
    blk = N_TOKENS // grid
    return pl.pallas_call(
        _combine_kernel,
        grid=(grid,),
        in_specs=[
            pl.BlockSpec((blk, D_MODEL), lambda i: (i, 0)),
            pl.BlockSpec((blk, D_MODEL), lambda i: (i, 0)),
            pl.BlockSpec((blk, 1), lambda i: (i, 0)),
            pl.BlockSpec((blk, 1), lambda i: (i, 0)),
        ],
        out_specs=pl.BlockSpec((blk, D_MODEL), lambda i: (i, 0)),
        out_shape=jax.ShapeDtypeStruct((N_TOKENS, D_MODEL), jnp.float32),
    )(y0, y1, g1, g2)


def kernel(x, gate_W, gate_b, expert_W, expert_b):
    e1, e2, r0, r1, g1, g2, counts = _routing(x, gate_W, gate_b)

    counts = counts.reshape(NUM_EXPERTS)
    padded = (counts + TILE - 1) // TILE * TILE
    off = jnp.concatenate([jnp.zeros((1,), jnp.int32),
                           jnp.cumsum(padded)[:-1].astype(jnp.int32)])
    off16 = jnp.concatenate([off, jnp.zeros((16 - NUM_EXPERTS,), jnp.int32)])
    tile_starts = jnp.arange(NUM_TILES, dtype=jnp.int32) * TILE
    tile_expert = (jnp.sum((tile_starts[:, None] >= off[None, :])
                           .astype(jnp.int32), axis=1) - 1)
    tile_expert = jnp.clip(tile_expert, 0, NUM_EXPERTS - 1)

    expert_W_bf16 = expert_W.astype(jnp.bfloat16)
    xs = _scatter_fn()(x, e1, e2, r0, r1, off16)
    y = _gmm(tile_expert, xs, expert_W_bf16, expert_b)
    y0, y1 = _gather2_fn()(y, e1, e2, r0, r1, off16)
    return _tc_combine(y0, y1, g1, g2)
